# baseline (speedup 1.0000x reference)
import functools

import jax
import jax.numpy as jnp
from jax import lax
from jax.experimental import pallas as pl
from jax.experimental.pallas import tpu as pltpu
from jax.experimental.pallas import tpu_sc as plsc

NUM_CORES = 2
NUM_SUBCORES = 16
NUM_WORKERS = NUM_CORES * NUM_SUBCORES
IDX_CHUNK = 128


def _sc_gather(table, heads, tails):
    B = heads.shape[0]
    D = table.shape[1]
    b_per_w = B // NUM_WORKERS
    n_chunks = b_per_w // IDX_CHUNK
    mesh = plsc.VectorSubcoreMesh(
        core_axis_name="c", subcore_axis_name="s",
        num_cores=NUM_CORES, num_subcores=NUM_SUBCORES)

    @functools.partial(
        pl.kernel,
        mesh=mesh,
        compiler_params=pltpu.CompilerParams(use_tc_tiling_on_sc=False),
        out_type=(
            jax.ShapeDtypeStruct((B, D), jnp.float32),
            jax.ShapeDtypeStruct((B, D), jnp.float32),
        ),
        scratch_types=[
            pltpu.VMEM((n_chunks, IDX_CHUNK), jnp.int32),
            pltpu.VMEM((n_chunks, IDX_CHUNK), jnp.int32),
            pltpu.VMEM((b_per_w, D), jnp.float32),
            pltpu.VMEM((b_per_w, D), jnp.float32),
            pltpu.SemaphoreType.DMA,
            pltpu.SemaphoreType.DMA,
        ],
    )
    def gather_kernel(table_hbm, heads_hbm, tails_hbm, outh_hbm, outt_hbm,
                      hidx_v, tidx_v, hrows_v, trows_v, hsem, tsem):
        wid = lax.axis_index("s") * NUM_CORES + lax.axis_index("c")
        base = wid * b_per_w
        for c in range(n_chunks):
            pltpu.sync_copy(heads_hbm.at[pl.ds(base + c * IDX_CHUNK, IDX_CHUNK)],
                            hidx_v.at[c])
            pltpu.sync_copy(tails_hbm.at[pl.ds(base + c * IDX_CHUNK, IDX_CHUNK)],
                            tidx_v.at[c])
        copies = []
        for c in range(n_chunks):
            copies.append(pltpu.async_copy(
                table_hbm.at[hidx_v.at[c]],
                hrows_v.at[pl.ds(c * IDX_CHUNK, IDX_CHUNK)], hsem))
            copies.append(pltpu.async_copy(
                table_hbm.at[tidx_v.at[c]],
                trows_v.at[pl.ds(c * IDX_CHUNK, IDX_CHUNK)], tsem))
        for cp in copies:
            cp.wait()
        pltpu.sync_copy(hrows_v, outh_hbm.at[pl.ds(base, b_per_w)])
        pltpu.sync_copy(trows_v, outt_hbm.at[pl.ds(base, b_per_w)])

    return gather_kernel(table, heads, tails)


def _tc_distance(head_rows, tail_rows, W, b):
    B, D = head_rows.shape
    BLK = 2048
    grid = (B // BLK,)

    def body(h_ref, t_ref, w_ref, b_ref, o_ref):
        y = jnp.dot(h_ref[...], w_ref[...].T,
                    preferred_element_type=jnp.float32)
        y = y + b_ref[...] - t_ref[...]
        d = jnp.sqrt(jnp.sum(y * y, axis=1))
        o_ref[...] = -d[None, :]

    out = pl.pallas_call(
        body,
        grid=grid,
        in_specs=[
            pl.BlockSpec((BLK, D), lambda i: (i, 0)),
            pl.BlockSpec((BLK, D), lambda i: (i, 0)),
            pl.BlockSpec((D, D), lambda i: (0, 0)),
            pl.BlockSpec((1, D), lambda i: (0, 0)),
        ],
        out_specs=pl.BlockSpec((1, BLK), lambda i: (0, i)),
        out_shape=jax.ShapeDtypeStruct((1, B), jnp.float32),
    )(head_rows, tail_rows, W, b.reshape(1, D))
    return out.reshape(B)


def kernel(heads, tails, entity_embeddings, W, b):
    head_rows, tail_rows = _sc_gather(entity_embeddings, heads, tails)
    return _tc_distance(head_rows, tail_rows, W, b)


# speedup vs baseline: 1.4945x; 1.4945x over previous
import functools

import jax
import jax.numpy as jnp
from jax import lax
from jax.experimental import pallas as pl
from jax.experimental.pallas import tpu as pltpu
from jax.experimental.pallas import tpu_sc as plsc

NUM_CORES = 2
NUM_SUBCORES = 16
NUM_WORKERS = NUM_CORES * NUM_SUBCORES


def _sc_gather(table, heads, tails):
    B = heads.shape[0]
    D = table.shape[1]
    b_per_w = B // NUM_WORKERS
    row_bytes = D * 4
    mesh = plsc.VectorSubcoreMesh(
        core_axis_name="c", subcore_axis_name="s",
        num_cores=NUM_CORES, num_subcores=NUM_SUBCORES)

    @functools.partial(
        pl.kernel,
        mesh=mesh,
        out_type=(
            jax.ShapeDtypeStruct((B, D), jnp.float32),
            jax.ShapeDtypeStruct((B, D), jnp.float32),
        ),
        scratch_types=[
            pltpu.VMEM((b_per_w,), jnp.int32),
            pltpu.VMEM((b_per_w,), jnp.int32),
            pltpu.SemaphoreType.DMA,
        ],
    )
    def gather_kernel(table_hbm, heads_hbm, tails_hbm, outh_hbm, outt_hbm,
                      hidx_v, tidx_v, sem):
        wid = lax.axis_index("s") * NUM_CORES + lax.axis_index("c")
        base = wid * b_per_w
        pltpu.sync_copy(heads_hbm.at[pl.ds(base, b_per_w)], hidx_v)
        pltpu.sync_copy(tails_hbm.at[pl.ds(base, b_per_w)], tidx_v)

        def body(g, carry):
            hvec = hidx_v[pl.ds(g * 16, 16)]
            tvec = tidx_v[pl.ds(g * 16, 16)]
            for k in range(16):
                j = base + g * 16 + k
                pltpu.async_copy(table_hbm.at[pl.ds(hvec[k], 1), :],
                                 outh_hbm.at[pl.ds(j, 1), :], sem)
                pltpu.async_copy(table_hbm.at[pl.ds(tvec[k], 1), :],
                                 outt_hbm.at[pl.ds(j, 1), :], sem)
            return carry

        lax.fori_loop(0, b_per_w // 16, body, 0)

        def drain(j, carry):
            pltpu.make_async_copy(table_hbm.at[pl.ds(0, 1), :],
                                  outh_hbm.at[pl.ds(0, 1), :], sem).wait()
            return carry

        lax.fori_loop(0, 2 * b_per_w, drain, 0)

    return gather_kernel(table, heads, tails)


def _tc_distance(head_rows, tail_rows, W, b):
    B, D = head_rows.shape
    BLK = 2048
    grid = (B // BLK,)

    def body(h_ref, t_ref, w_ref, b_ref, o_ref):
        y = jnp.dot(h_ref[...], w_ref[...].T,
                    preferred_element_type=jnp.float32)
        y = y + b_ref[...] - t_ref[...]
        d = jnp.sqrt(jnp.sum(y * y, axis=1))
        o_ref[...] = -d[None, :]

    out = pl.pallas_call(
        body,
        grid=grid,
        in_specs=[
            pl.BlockSpec((BLK, D), lambda i: (i, 0)),
            pl.BlockSpec((BLK, D), lambda i: (i, 0)),
            pl.BlockSpec((D, D), lambda i: (0, 0)),
            pl.BlockSpec((1, D), lambda i: (0, 0)),
        ],
        out_specs=pl.BlockSpec((1, BLK), lambda i: (0, i)),
        out_shape=jax.ShapeDtypeStruct((1, B), jnp.float32),
    )(head_rows, tail_rows, W, b.reshape(1, D))
    return out.reshape(B)


def kernel(heads, tails, entity_embeddings, W, b):
    head_rows, tail_rows = _sc_gather(entity_embeddings, heads, tails)
    return _tc_distance(head_rows, tail_rows, W, b)
